# baseline (speedup 1.0000x reference)
import functools

import jax
import jax.numpy as jnp
from jax import lax
from jax.experimental import pallas as pl
from jax.experimental.pallas import tpu as pltpu
from jax.experimental.pallas import tpu_sc as plsc

N_VOCAB = 1000000
N_DENSE = 13
N_SPARSE = 26
D_EMB = 64

NC = 2
NS = 16
NW = NC * NS
CH = 64
NBUF = 8


def _gather_body(table_hbm, idx_hbm, out_hbm, idx_v, rows_v, gsem, nch):
    wid = lax.axis_index("s") * NC + lax.axis_index("c")
    out_base = wid * (nch * CH)

    pltpu.sync_copy(idx_hbm.at[wid], idx_v)

    for b in range(NBUF):
        pltpu.async_copy(table_hbm.at[idx_v.at[b]], rows_v.at[b], gsem)

    n_outer = nch // NBUF

    def outer(g, _):
        for b in range(NBUF):
            j = g * NBUF + b
            pltpu.make_async_copy(
                table_hbm.at[idx_v.at[b]], rows_v.at[b], gsem
            ).wait()
            pltpu.sync_copy(
                rows_v.at[b], out_hbm.at[pl.ds(out_base + j * CH, CH)]
            )

            @pl.when(g + 1 < n_outer)
            def _():
                pltpu.async_copy(
                    table_hbm.at[idx_v.at[j + NBUF]], rows_v.at[b], gsem
                )

        return ()

    lax.fori_loop(0, n_outer, outer, (), unroll=False)


def _sc_gather(table_pad, idx):
    _, nch, _ = idx.shape
    n = NW * nch * CH
    mesh = plsc.VectorSubcoreMesh(
        core_axis_name="c", subcore_axis_name="s", num_cores=NC,
        num_subcores=NS,
    )
    kern = pl.kernel(
        functools.partial(_gather_body, nch=nch),
        out_type=jax.ShapeDtypeStruct((n, 2 * D_EMB), jnp.float32),
        mesh=mesh,
        scratch_types=[
            pltpu.VMEM((nch, CH), jnp.int32),
            pltpu.VMEM((NBUF, CH, 2 * D_EMB), jnp.float32),
            pltpu.SemaphoreType.DMA,
        ],
        compiler_params=pltpu.CompilerParams(use_tc_tiling_on_sc=False),
    )
    return kern(table_pad, idx)


def _tpose_body(tt_ref, eye_ref, out_ref):
    out_ref[...] = lax.dot_general(
        tt_ref[...], eye_ref[...], (((0,), (0,)), ((), ())),
        preferred_element_type=jnp.float32)


def _tc_transpose(tt, eye_pad, bk=16384):
    v = tt.shape[1]
    grid = ((v + bk - 1) // bk,)
    return pl.pallas_call(
        _tpose_body,
        grid=grid,
        in_specs=[pl.BlockSpec((D_EMB, bk), lambda i: (0, i)),
                  pl.BlockSpec((D_EMB, 2 * D_EMB), lambda i: (0, 0))],
        out_specs=pl.BlockSpec((bk, 2 * D_EMB), lambda i: (i, 0)),
        out_shape=jax.ShapeDtypeStruct((v, 2 * D_EMB), jnp.float32),
    )(tt, eye_pad)


def _mlp_body(dense_ref, emb_ref,
              wb0_ref, bb0_ref, wb1_ref, bb1_ref, wb2_ref, bb2_ref,
              w0d_ref, w0e_ref, bt0_ref, wt1_ref, bt1_ref,
              wt2_ref, bt2_ref, wt3_ref, bt3_ref, wo_ref, bo_ref,
              out_ref):
    f32 = jnp.float32
    bf = jnp.bfloat16

    def mm(a, b_ref):
        return jnp.dot(a.astype(bf), b_ref[...].astype(bf),
                       preferred_element_type=f32)

    d = dense_ref[...]
    bot = jax.nn.relu(jnp.dot(d, wb0_ref[...], preferred_element_type=f32)
                      + bb0_ref[...])
    bot = bot + jax.nn.relu(mm(bot, wb1_ref) + bb1_ref[...])
    bot = bot + jax.nn.relu(mm(bot, wb2_ref) + bb2_ref[...])

    acc = mm(bot, w0d_ref) + bt0_ref[...]
    for s in range(N_SPARSE):
        acc = acc + jnp.dot(emb_ref[s][:, :D_EMB].astype(bf),
                            w0e_ref[s].astype(bf),
                            preferred_element_type=f32)
    top = jax.nn.relu(acc)
    top = top + jax.nn.relu(mm(top, wt1_ref) + bt1_ref[...])
    top = top + jax.nn.relu(mm(top, wt2_ref) + bt2_ref[...])
    top = top + jax.nn.relu(mm(top, wt3_ref) + bt3_ref[...])
    out_ref[...] = (jnp.dot(top, wo_ref[...], preferred_element_type=f32)
                    + bo_ref[...])


def _tc_mlp(dense, emb, W_bot0, b_bot0, W_bot1, b_bot1, W_bot2, b_bot2,
            W0d, W0e, b_top0, W_top1, b_top1, W_top2, b_top2,
            W_top3, b_top3, W_out, b_out, block_rows):
    batch = dense.shape[0]
    grid = (batch // block_rows,)

    def row_spec(cols):
        return pl.BlockSpec((block_rows, cols), lambda i: (i, 0))

    def full_spec(a):
        return pl.BlockSpec(a.shape, lambda i: (0,) * a.ndim)

    emb_spec = pl.BlockSpec((N_SPARSE, block_rows, 128),
                            lambda i: (0, i, 0))

    weights = (W_bot0, b_bot0, W_bot1, b_bot1, W_bot2, b_bot2,
               W0d, W0e, b_top0, W_top1, b_top1, W_top2, b_top2,
               W_top3, b_top3, W_out, b_out)

    return pl.pallas_call(
        _mlp_body,
        grid=grid,
        in_specs=[row_spec(N_DENSE), emb_spec]
                 + [full_spec(w) for w in weights],
        out_specs=row_spec(1),
        out_shape=jax.ShapeDtypeStruct((batch, 1), jnp.float32),
    )(dense, emb, *weights)


def kernel(x, W_bot0, b_bot0, W_bot1, b_bot1, W_bot2, b_bot2, emb_table,
           W_top0, b_top0, W_top1, b_top1, W_top2, b_top2, W_top3, b_top3,
           W_out, b_out):
    batch = x.shape[0]
    dense = x[:, :N_DENSE]
    n = batch * N_SPARSE
    per_w = n // NW
    nch = per_w // CH
    cat = x[:, N_DENSE:].astype(jnp.int32) % N_VOCAB
    idx = cat.T.reshape(NW, nch, CH)
    eye_pad = jnp.eye(D_EMB, 2 * D_EMB, dtype=jnp.float32)
    table_pad = _tc_transpose(emb_table.T, eye_pad)

    emb = _sc_gather(table_pad, idx).reshape(N_SPARSE, batch, 2 * D_EMB)

    W0d = W_top0[:256]
    W0e = W_top0[256:].reshape(N_SPARSE, D_EMB, 256)
    row = lambda v: v.reshape(1, -1)
    return _tc_mlp(
        dense, emb, W_bot0, row(b_bot0), W_bot1, row(b_bot1), W_bot2,
        row(b_bot2), W0d, W0e, row(b_top0), W_top1, row(b_top1), W_top2,
        row(b_top2), W_top3, row(b_top3), W_out, row(b_out),
        block_rows=1024)


# speedup vs baseline: 1.1408x; 1.1408x over previous
import functools

import jax
import jax.numpy as jnp
from jax import lax
from jax.experimental import pallas as pl
from jax.experimental.pallas import tpu as pltpu
from jax.experimental.pallas import tpu_sc as plsc

N_VOCAB = 1000000
N_DENSE = 13
N_SPARSE = 26
D_EMB = 64

NC = 2
NS = 16
NW = NC * NS
CH = 128
NBUF = 4


def _gather_body(table_hbm, idx_hbm, out_hbm, catv, rows_v, gsem,
                 nch, batch):
    wid = lax.axis_index("s") * NC + lax.axis_index("c")
    per_w = nch * CH
    slab = 2 * batch
    runlen = per_w // 2

    d0 = wid * per_w
    k0 = d0 // slab
    k1 = (d0 + per_w - 1) // slab
    b00 = (d0 - k0 * slab) // 2

    al = lambda v: pl.multiple_of(v, 8)
    pltpu.sync_copy(idx_hbm.at[pl.ds(al(k0 * slab + b00), runlen)],
                    catv.at[0])
    pltpu.sync_copy(idx_hbm.at[pl.ds(al(k0 * slab + batch + b00), runlen)],
                    catv.at[1])
    pltpu.sync_copy(idx_hbm.at[pl.ds(al(k1 * slab), runlen)], catv.at[2])
    pltpu.sync_copy(idx_hbm.at[pl.ds(al(k1 * slab + batch), runlen)],
                    catv.at[3])

    half = CH // 2

    def fire(j, slot):
        d = d0 + j * CH
        k = d // slab
        colbase = (d - k * slab) // 2 - jnp.where(k == k0, b00, 0)
        colbase = pl.multiple_of(colbase, 8)
        rsel = jnp.where(k == k0, 0, 2)
        idx0 = catv.at[rsel, pl.ds(colbase, half)]
        idx1 = catv.at[rsel + 1, pl.ds(colbase, half)]
        pltpu.async_copy(table_hbm.at[idx0], rows_v.at[slot, 0], gsem)
        pltpu.async_copy(table_hbm.at[idx1], rows_v.at[slot, 1], gsem)

    def drain(slot):
        for h in range(2):
            pltpu.make_async_copy(
                table_hbm.at[catv.at[0, pl.ds(0, half)]],
                rows_v.at[slot, h], gsem
            ).wait()

    for b in range(NBUF):
        fire(b, b)

    n_outer = nch // NBUF

    def outer(g, _):
        for b in range(NBUF):
            j = g * NBUF + b
            drain(b)
            rb = d0 // 2 + j * half
            pltpu.sync_copy(
                rows_v.at[b, 0].at[:, pl.ds(0, D_EMB)],
                out_hbm.at[pl.ds(rb, half), pl.ds(0, D_EMB)])
            pltpu.sync_copy(
                rows_v.at[b, 1].at[:, pl.ds(0, D_EMB)],
                out_hbm.at[pl.ds(rb, half), pl.ds(D_EMB, D_EMB)])

            @pl.when(g + 1 < n_outer)
            def _():
                fire(j + NBUF, b)

        return ()

    lax.fori_loop(0, n_outer, outer, (), unroll=False)


def _sc_gather(table_pad, idx_fm, nch, batch):
    n = idx_fm.shape[0]
    mesh = plsc.VectorSubcoreMesh(
        core_axis_name="c", subcore_axis_name="s", num_cores=NC,
        num_subcores=NS,
    )
    kern = pl.kernel(
        functools.partial(_gather_body, nch=nch, batch=batch),
        out_type=jax.ShapeDtypeStruct((n // 2, 2 * D_EMB), jnp.float32),
        mesh=mesh,
        scratch_types=[
            pltpu.VMEM((4, (nch * CH) // 2), jnp.int32),
            pltpu.VMEM((NBUF, 2, CH // 2, 2 * D_EMB), jnp.float32),
            pltpu.SemaphoreType.DMA,
        ],
        compiler_params=pltpu.CompilerParams(use_tc_tiling_on_sc=False),
    )
    return kern(table_pad, idx_fm)


def _tpose_body(tt_ref, eye_ref, out_ref):
    out_ref[...] = lax.dot_general(
        tt_ref[...], eye_ref[...], (((0,), (0,)), ((), ())),
        preferred_element_type=jnp.float32)


def _tc_transpose(tt, eye_pad, bk=16384):
    v = tt.shape[1]
    grid = ((v + bk - 1) // bk,)
    return pl.pallas_call(
        _tpose_body,
        grid=grid,
        in_specs=[pl.BlockSpec((D_EMB, bk), lambda i: (0, i)),
                  pl.BlockSpec((D_EMB, 2 * D_EMB), lambda i: (0, 0))],
        out_specs=pl.BlockSpec((bk, 2 * D_EMB), lambda i: (i, 0)),
        out_shape=jax.ShapeDtypeStruct((v, 2 * D_EMB), jnp.float32),
    )(tt, eye_pad)


def _mlp_body(dense_ref, emb_ref,
              wb0_ref, bb0_ref, wb1_ref, bb1_ref, wb2_ref, bb2_ref,
              w0d_ref, w0e_ref, bt0_ref, wt1_ref, bt1_ref,
              wt2_ref, bt2_ref, wt3_ref, bt3_ref, wo_ref, bo_ref,
              out_ref):
    f32 = jnp.float32
    bf = jnp.bfloat16

    def mm(a, b_ref):
        return jnp.dot(a.astype(bf), b_ref[...].astype(bf),
                       preferred_element_type=f32)

    d = dense_ref[...]
    bot = jax.nn.relu(jnp.dot(d, wb0_ref[...], preferred_element_type=f32)
                      + bb0_ref[...])
    bot = bot + jax.nn.relu(mm(bot, wb1_ref) + bb1_ref[...])
    bot = bot + jax.nn.relu(mm(bot, wb2_ref) + bb2_ref[...])

    acc = mm(bot, w0d_ref) + bt0_ref[...]
    for kk in range(N_SPARSE // 2):
        acc = acc + jnp.dot(emb_ref[kk].astype(bf), w0e_ref[kk].astype(bf),
                            preferred_element_type=f32)
    top = jax.nn.relu(acc)
    top = top + jax.nn.relu(mm(top, wt1_ref) + bt1_ref[...])
    top = top + jax.nn.relu(mm(top, wt2_ref) + bt2_ref[...])
    top = top + jax.nn.relu(mm(top, wt3_ref) + bt3_ref[...])
    out_ref[...] = (jnp.dot(top, wo_ref[...], preferred_element_type=f32)
                    + bo_ref[...])


def _tc_mlp(dense, emb, W_bot0, b_bot0, W_bot1, b_bot1, W_bot2, b_bot2,
            W0d, W0e, b_top0, W_top1, b_top1, W_top2, b_top2,
            W_top3, b_top3, W_out, b_out, block_rows):
    batch = dense.shape[0]
    grid = (batch // block_rows,)

    def row_spec(cols):
        return pl.BlockSpec((block_rows, cols), lambda i: (i, 0))

    def full_spec(a):
        return pl.BlockSpec(a.shape, lambda i: (0,) * a.ndim)

    emb_spec = pl.BlockSpec((N_SPARSE // 2, block_rows, 128),
                            lambda i: (0, i, 0))

    weights = (W_bot0, b_bot0, W_bot1, b_bot1, W_bot2, b_bot2,
               W0d, W0e, b_top0, W_top1, b_top1, W_top2, b_top2,
               W_top3, b_top3, W_out, b_out)

    return pl.pallas_call(
        _mlp_body,
        grid=grid,
        in_specs=[row_spec(N_DENSE), emb_spec]
                 + [full_spec(w) for w in weights],
        out_specs=row_spec(1),
        out_shape=jax.ShapeDtypeStruct((batch, 1), jnp.float32),
    )(dense, emb, *weights)


def kernel(x, W_bot0, b_bot0, W_bot1, b_bot1, W_bot2, b_bot2, emb_table,
           W_top0, b_top0, W_top1, b_top1, W_top2, b_top2, W_top3, b_top3,
           W_out, b_out):
    batch = x.shape[0]
    dense = x[:, :N_DENSE]
    n = batch * N_SPARSE
    per_w = n // NW
    nch = per_w // CH
    cat = x[:, N_DENSE:].astype(jnp.int32) % N_VOCAB
    idx_fm = cat.T.reshape(-1)
    eye_pad = jnp.eye(D_EMB, 2 * D_EMB, dtype=jnp.float32)
    table_pad = _tc_transpose(emb_table.T, eye_pad)

    emb = _sc_gather(table_pad, idx_fm, nch, batch)
    emb = emb.reshape(N_SPARSE // 2, batch, 128)

    W0d = W_top0[:256]
    W0e = W_top0[256:].reshape(N_SPARSE // 2, 128, 256)
    row = lambda v: v.reshape(1, -1)
    return _tc_mlp(
        dense, emb, W_bot0, row(b_bot0), W_bot1, row(b_bot1), W_bot2,
        row(b_bot2), W0d, W0e, row(b_top0), W_top1, row(b_top1), W_top2,
        row(b_top2), W_top3, row(b_top3), W_out, row(b_out),
        block_rows=1024)
